# Initial kernel scaffold; baseline (speedup 1.0000x reference)
import jax
import jax.numpy as jnp
from jax.experimental import pallas as pl


def kernel(input_tensor, Wg, bg, W1, b1, W2, b2):
    raise NotImplementedError("write your pallas kernel here")


# speedup vs baseline: 2.2151x; 2.2151x over previous
import functools

import jax
import jax.numpy as jnp
from jax.experimental import pallas as pl
from jax.experimental.pallas import tpu as pltpu

B, S, D, F, E, K = 1, 2048, 1024, 2048, 8, 2
ST = 512
T = S // ST

_INTERPRET = False


def _moe_body(x_ref, wg_ref, bg_ref, w1_ref, b1_ref, w2_ref, b2_ref,
              out_ref, wmask_ref, vacc_ref, oacc_ref, tw_ref, wsum_ref):
    e = pl.program_id(0)
    t = pl.program_id(1)
    x = x_ref[...]

    @pl.when(e == 0)
    def _router():
        scores = jnp.dot(x, wg_ref[...], preferred_element_type=jnp.float32)
        scores = scores + bg_ref[...]
        m1 = jnp.max(scores, axis=1, keepdims=True)
        i1 = jnp.argmax(scores, axis=1).astype(jnp.int32)
        col = jax.lax.broadcasted_iota(jnp.int32, scores.shape, 1)
        sel1 = col == i1[:, None]
        masked = jnp.where(sel1, -jnp.inf, scores)
        m2 = jnp.max(masked, axis=1, keepdims=True)
        i2 = jnp.argmax(masked, axis=1).astype(jnp.int32)
        sel2 = col == i2[:, None]
        e2 = jnp.exp(m2 - m1)
        denom = 1.0 + e2
        wm = jnp.where(sel1, 1.0 / denom, 0.0) + jnp.where(sel2, e2 / denom, 0.0)
        wmask_ref[:, pl.ds(t * ST, ST)] = wm.T
        @pl.when(t == 0)
        def _():
            tw_ref[0] = 0.0
        tw_ref[0] += jnp.sum(wm)

    @pl.when(t == 0)
    def _():
        vacc_ref[...] = jnp.zeros_like(vacc_ref)
        wsum_ref[0] = 0.0

    w_row = wmask_ref[pl.ds(e, 1), pl.ds(t * ST, ST)]
    h = jnp.dot(x, w1_ref[0], preferred_element_type=jnp.float32)
    h = jnp.maximum(h + b1_ref[0], 0.0)
    vacc_ref[...] += jnp.dot(w_row, h, preferred_element_type=jnp.float32)
    wsum_ref[0] += jnp.sum(w_row)

    @pl.when(t == T - 1)
    def _finish_expert():
        contrib = jnp.dot(vacc_ref[...], w2_ref[0],
                          preferred_element_type=jnp.float32)
        contrib = contrib + wsum_ref[0] * b2_ref[0]
        @pl.when(e == 0)
        def _():
            oacc_ref[...] = jnp.zeros_like(oacc_ref)
        oacc_ref[...] += contrib
        @pl.when(e == E - 1)
        def _():
            out_ref[...] = oacc_ref[...] / tw_ref[0]


@functools.partial(jax.jit, static_argnames=())
def kernel(input_tensor, Wg, bg, W1, b1, W2, b2):
    x = input_tensor.reshape(S, D)
    out = pl.pallas_call(
        _moe_body,
        grid=(E, T),
        in_specs=[
            pl.BlockSpec((ST, D), lambda e, t: (t, 0)),
            pl.BlockSpec((D, E), lambda e, t: (0, 0)),
            pl.BlockSpec((1, E), lambda e, t: (0, 0)),
            pl.BlockSpec((1, D, F), lambda e, t: (e, 0, 0)),
            pl.BlockSpec((1, 1, F), lambda e, t: (e, 0, 0)),
            pl.BlockSpec((1, F, D), lambda e, t: (e, 0, 0)),
            pl.BlockSpec((1, 1, D), lambda e, t: (e, 0, 0)),
        ],
        out_specs=pl.BlockSpec((1, D), lambda e, t: (0, 0)),
        out_shape=jax.ShapeDtypeStruct((1, D), jnp.float32),
        scratch_shapes=[
            pltpu.VMEM((E, S), jnp.float32),
            pltpu.VMEM((1, F), jnp.float32),
            pltpu.VMEM((1, D), jnp.float32),
            pltpu.SMEM((1,), jnp.float32),
            pltpu.SMEM((1,), jnp.float32),
        ],
        compiler_params=pltpu.CompilerParams(
            dimension_semantics=("arbitrary", "arbitrary"),
        ),
        interpret=_INTERPRET,
    )(x, Wg, bg.reshape(1, E), W1, b1.reshape(E, 1, F), W2, b2.reshape(E, 1, D))
    return out.reshape(1, 1, D)


# speedup vs baseline: 2.2197x; 1.0021x over previous
import functools

import jax
import jax.numpy as jnp
from jax.experimental import pallas as pl
from jax.experimental.pallas import tpu as pltpu

B, S, D, F, E, K = 1, 2048, 1024, 2048, 8, 2
ST = 512
T = S // ST

_INTERPRET = False


def _moe_body(x_ref, wg_ref, bg_ref, w1_ref, b1_ref, w2_ref, b2_ref,
              out_ref, wmask_ref, vacc_ref, oacc_ref, tw_ref, wsum_ref):
    e = pl.program_id(0)
    t = pl.program_id(1)
    x = x_ref[...]

    @pl.when(e == 0)
    def _router():
        scores = jnp.dot(x, wg_ref[...], preferred_element_type=jnp.float32)
        scores = scores + bg_ref[...]
        m1 = jnp.max(scores, axis=1, keepdims=True)
        i1 = jnp.argmax(scores, axis=1).astype(jnp.int32)
        col = jax.lax.broadcasted_iota(jnp.int32, scores.shape, 1)
        sel1 = col == i1[:, None]
        masked = jnp.where(sel1, -jnp.inf, scores)
        m2 = jnp.max(masked, axis=1, keepdims=True)
        i2 = jnp.argmax(masked, axis=1).astype(jnp.int32)
        sel2 = col == i2[:, None]
        e2 = jnp.exp(m2 - m1)
        denom = 1.0 + e2
        wm = jnp.where(sel1, 1.0 / denom, 0.0) + jnp.where(sel2, e2 / denom, 0.0)
        wmask_ref[:, pl.ds(t * ST, ST)] = wm.T
        @pl.when(t == 0)
        def _():
            tw_ref[0] = 0.0
        tw_ref[0] += jnp.sum(wm)

    @pl.when(t == 0)
    def _():
        vacc_ref[...] = jnp.zeros_like(vacc_ref)
        wsum_ref[0] = 0.0

    w_row = wmask_ref[pl.ds(e, 1), pl.ds(t * ST, ST)]
    h = jnp.dot(x.astype(jnp.bfloat16), w1_ref[0].astype(jnp.bfloat16),
                preferred_element_type=jnp.float32)
    h = jnp.maximum(h + b1_ref[0], 0.0)
    vacc_ref[...] += jnp.dot(w_row, h, preferred_element_type=jnp.float32)
    wsum_ref[0] += jnp.sum(w_row)

    @pl.when(t == T - 1)
    def _finish_expert():
        contrib = jnp.dot(vacc_ref[...], w2_ref[0],
                          preferred_element_type=jnp.float32)
        contrib = contrib + wsum_ref[0] * b2_ref[0]
        @pl.when(e == 0)
        def _():
            oacc_ref[...] = jnp.zeros_like(oacc_ref)
        oacc_ref[...] += contrib
        @pl.when(e == E - 1)
        def _():
            out_ref[...] = oacc_ref[...] / tw_ref[0]


@functools.partial(jax.jit, static_argnames=())
def kernel(input_tensor, Wg, bg, W1, b1, W2, b2):
    x = input_tensor.reshape(S, D)
    out = pl.pallas_call(
        _moe_body,
        grid=(E, T),
        in_specs=[
            pl.BlockSpec((ST, D), lambda e, t: (t, 0)),
            pl.BlockSpec((D, E), lambda e, t: (0, 0)),
            pl.BlockSpec((1, E), lambda e, t: (0, 0)),
            pl.BlockSpec((1, D, F), lambda e, t: (e, 0, 0)),
            pl.BlockSpec((1, 1, F), lambda e, t: (e, 0, 0)),
            pl.BlockSpec((1, F, D), lambda e, t: (e, 0, 0)),
            pl.BlockSpec((1, 1, D), lambda e, t: (e, 0, 0)),
        ],
        out_specs=pl.BlockSpec((1, D), lambda e, t: (0, 0)),
        out_shape=jax.ShapeDtypeStruct((1, D), jnp.float32),
        scratch_shapes=[
            pltpu.VMEM((E, S), jnp.float32),
            pltpu.VMEM((1, F), jnp.float32),
            pltpu.VMEM((1, D), jnp.float32),
            pltpu.SMEM((1,), jnp.float32),
            pltpu.SMEM((1,), jnp.float32),
        ],
        compiler_params=pltpu.CompilerParams(
            dimension_semantics=("arbitrary", "arbitrary"),
        ),
        interpret=_INTERPRET,
    )(x, Wg, bg.reshape(1, E), W1, b1.reshape(E, 1, F), W2, b2.reshape(E, 1, D))
    return out.reshape(1, 1, D)
